# Initial kernel scaffold; baseline (speedup 1.0000x reference)
import jax
import jax.numpy as jnp
from jax.experimental import pallas as pl


def kernel(input_features, part_cls, obj_cls, W):
    raise NotImplementedError("write your pallas kernel here")


# speedup vs baseline: 1.0306x; 1.0306x over previous
import jax
import jax.numpy as jnp
from jax.experimental import pallas as pl
from jax.experimental.pallas import tpu as pltpu

_ROWS = 16384
_COLS = 512
_BLOCK_ROWS = 2048


def _scale_kernel(p_ref, o_ref, w_ref, x_ref, out_ref, s_ref):
    @pl.when(pl.program_id(0) == 0)
    def _():
        rows = jax.lax.broadcasted_iota(jnp.int32, (95, 95), 0)
        cols = jax.lax.broadcasted_iota(jnp.int32, (95, 95), 1)
        hit = (rows == p_ref[0]) & (cols == o_ref[0])
        w = jnp.sum(jnp.where(hit, w_ref[...], 0.0))
        s_ref[0] = jax.nn.sigmoid(w)

    out_ref[...] = x_ref[...] * s_ref[0]


def kernel(input_features, part_cls, obj_cls, W):
    p = jnp.asarray(part_cls, jnp.int32).reshape(1)
    o = jnp.asarray(obj_cls, jnp.int32).reshape(1)
    w2d = W.reshape(95, 95)
    grid = _ROWS // _BLOCK_ROWS
    return pl.pallas_call(
        _scale_kernel,
        grid_spec=pltpu.PrefetchScalarGridSpec(
            num_scalar_prefetch=2,
            grid=(grid,),
            in_specs=[
                pl.BlockSpec((95, 95), lambda i, p, o: (0, 0)),
                pl.BlockSpec((_BLOCK_ROWS, _COLS), lambda i, p, o: (i, 0)),
            ],
            out_specs=pl.BlockSpec((_BLOCK_ROWS, _COLS), lambda i, p, o: (i, 0)),
            scratch_shapes=[pltpu.SMEM((1,), jnp.float32)],
        ),
        out_shape=jax.ShapeDtypeStruct((_ROWS, _COLS), jnp.float32),
        compiler_params=pltpu.CompilerParams(
            dimension_semantics=("arbitrary",),
        ),
    )(p, o, w2d, input_features)


# speedup vs baseline: 1.0980x; 1.0654x over previous
import jax
import jax.numpy as jnp
from jax.experimental import pallas as pl
from jax.experimental.pallas import tpu as pltpu

_ROWS = 16384
_COLS = 512
_BLOCK_ROWS = 4096


def _scale_kernel(p_ref, o_ref, w_ref, x_ref, out_ref, s_ref):
    @pl.when(pl.program_id(0) == 0)
    def _():
        rows = jax.lax.broadcasted_iota(jnp.int32, (95, 95), 0)
        cols = jax.lax.broadcasted_iota(jnp.int32, (95, 95), 1)
        hit = (rows == p_ref[0]) & (cols == o_ref[0])
        w = jnp.sum(jnp.where(hit, w_ref[...], 0.0))
        s_ref[0] = jax.nn.sigmoid(w)

    out_ref[...] = x_ref[...] * s_ref[0]


def kernel(input_features, part_cls, obj_cls, W):
    p = jnp.asarray(part_cls, jnp.int32).reshape(1)
    o = jnp.asarray(obj_cls, jnp.int32).reshape(1)
    w2d = W.reshape(95, 95)
    grid = _ROWS // _BLOCK_ROWS
    return pl.pallas_call(
        _scale_kernel,
        grid_spec=pltpu.PrefetchScalarGridSpec(
            num_scalar_prefetch=2,
            grid=(grid,),
            in_specs=[
                pl.BlockSpec((95, 95), lambda i, p, o: (0, 0)),
                pl.BlockSpec((_BLOCK_ROWS, _COLS), lambda i, p, o: (i, 0)),
            ],
            out_specs=pl.BlockSpec((_BLOCK_ROWS, _COLS), lambda i, p, o: (i, 0)),
            scratch_shapes=[pltpu.SMEM((1,), jnp.float32)],
        ),
        out_shape=jax.ShapeDtypeStruct((_ROWS, _COLS), jnp.float32),
        compiler_params=pltpu.CompilerParams(
            dimension_semantics=("arbitrary",),
        ),
    )(p, o, w2d, input_features)
